# baseline (speedup 1.0000x reference)
import jax
import jax.numpy as jnp
from jax import lax
from jax.experimental import pallas as pl
from jax.experimental.pallas import tpu as pltpu
from jax.experimental.pallas import tpu_sc as plsc

_LENS = (480000, 448000, 416000, 384000, 352000, 320000, 288000, 256000)
_MAXL = 480000
_NC, _NS = 2, 16
_NW = _NC * _NS
_W = 16000
_NT = _MAXL // _W
_TPC = 32000 // _W


def _body(w0, w1, w2, w3, w4, w5, w6, w7, zrow, out, buf, isem, osem):
    ws = (w0, w1, w2, w3, w4, w5, w6, w7)
    wid = lax.axis_index("s") * _NC + lax.axis_index("c")
    t = wid
    valid = t < _NT
    c0 = pl.multiple_of(t * _W, _W)

    def in_copy(r):
        return pltpu.make_async_copy(
            ws[r].at[pl.ds(c0, _W)], buf.at[r], isem
        )

    def zero_copy(r):
        return pltpu.make_async_copy(zrow, buf.at[r], isem)

    for r in range(8):
        data = t < (15 - r) * _TPC

        @pl.when(valid & data)
        def _(r=r):
            in_copy(r).start()

        @pl.when(valid & jnp.logical_not(data))
        def _(r=r):
            zero_copy(r).start()

    @pl.when(valid)
    def _():
        pltpu.make_async_copy(out.at[:, pl.ds(0, _W)], buf, isem).wait()

    out_copy = pltpu.make_async_copy(
        buf, out.at[:, pl.ds(c0, _W)], osem
    )

    @pl.when(valid)
    def _():
        out_copy.start()
        out_copy.wait()


@jax.jit
def _pad_stack(w0, w1, w2, w3, w4, w5, w6, w7):
    mesh = plsc.VectorSubcoreMesh(core_axis_name="c", subcore_axis_name="s")
    f = pl.kernel(
        _body,
        out_type=jax.ShapeDtypeStruct((8, _MAXL), jnp.float32),
        mesh=mesh,
        scratch_types=[
            pltpu.VMEM((8, _W), jnp.float32),
            pltpu.SemaphoreType.DMA,
            pltpu.SemaphoreType.DMA,
        ],
    )
    zrow = jnp.zeros((_W,), jnp.float32)
    return f(w0, w1, w2, w3, w4, w5, w6, w7, zrow)


def kernel(w0, w1, w2, w3, w4, w5, w6, w7):
    batched = _pad_stack(w0, w1, w2, w3, w4, w5, w6, w7)
    wave_lengths = jnp.array(_LENS, dtype=jnp.int32)
    return (batched, wave_lengths)


# speedup vs baseline: 1.0003x; 1.0003x over previous
import jax
import jax.numpy as jnp
from jax import lax
from jax.experimental import pallas as pl
from jax.experimental.pallas import tpu as pltpu
from jax.experimental.pallas import tpu_sc as plsc

_LENS = (480000, 448000, 416000, 384000, 352000, 320000, 288000, 256000)
_MAXL = 480000
_NC, _NS = 2, 16
_NW = _NC * _NS
_W = 16000
_NT = _MAXL // _W
_TPC = 32000 // _W


def _body(w0, w1, w2, w3, w4, w5, w6, w7, zrow, out, buf, isem, osem):
    ws = (w0, w1, w2, w3, w4, w5, w6, w7)
    wid = lax.axis_index("s") * _NC + lax.axis_index("c")
    t = wid
    valid = t < _NT
    c0 = pl.multiple_of(t * _W, _W)

    def in_copy(r):
        return pltpu.make_async_copy(
            ws[r].at[pl.ds(c0, _W)], buf.at[r], isem
        )

    def zero_copy(r):
        return pltpu.make_async_copy(zrow, buf.at[r], isem)

    for r in range(8):
        data = t < (15 - r) * _TPC

        @pl.when(valid & data)
        def _(r=r):
            in_copy(r).start()

        @pl.when(valid & jnp.logical_not(data))
        def _(r=r):
            zero_copy(r).start()

    @pl.when(valid)
    def _():
        for r in range(8):
            in_copy(r).wait()

    out_copy = pltpu.make_async_copy(
        buf, out.at[:, pl.ds(c0, _W)], osem
    )

    @pl.when(valid)
    def _():
        out_copy.start()
        out_copy.wait()


@jax.jit
def _pad_stack(w0, w1, w2, w3, w4, w5, w6, w7):
    mesh = plsc.VectorSubcoreMesh(core_axis_name="c", subcore_axis_name="s")
    f = pl.kernel(
        _body,
        out_type=jax.ShapeDtypeStruct((8, _MAXL), jnp.float32),
        mesh=mesh,
        scratch_types=[
            pltpu.VMEM((8, _W), jnp.float32),
            pltpu.SemaphoreType.DMA,
            pltpu.SemaphoreType.DMA,
        ],
    )
    zrow = jnp.zeros((_W,), jnp.float32)
    return f(w0, w1, w2, w3, w4, w5, w6, w7, zrow)


def kernel(w0, w1, w2, w3, w4, w5, w6, w7):
    batched = _pad_stack(w0, w1, w2, w3, w4, w5, w6, w7)
    wave_lengths = jnp.array(_LENS, dtype=jnp.int32)
    return (batched, wave_lengths)
